# Initial kernel scaffold; baseline (speedup 1.0000x reference)
import jax
import jax.numpy as jnp
from jax.experimental import pallas as pl


def kernel(x, indices):
    raise NotImplementedError("write your pallas kernel here")


# speedup vs baseline: 1.5197x; 1.5197x over previous
import jax
import jax.numpy as jnp
from jax import lax
from jax.experimental import pallas as pl
from jax.experimental.pallas import tpu as pltpu
from jax.experimental.pallas import tpu_sc as plsc

B, S, D = 4, 4096, 1024
L = 2048
NC, NS = 2, 16
NW = NC * NS
RPW = (B * L) // NW
CH = 32
NCH = RPW // CH


def _body(x_hbm, idx_hbm, out_hbm, idx_v, buf0, buf1, gs0, gs1, ws0, ws1):
    wid = lax.axis_index("s") * NC + lax.axis_index("c")
    base = wid * RPW
    off = (base // L) * S

    pltpu.sync_copy(idx_hbm.at[pl.ds(base, RPW)], idx_v)
    for i in range(RPW // 16):
        v = idx_v[pl.ds(i * 16, 16)]
        idx_v[pl.ds(i * 16, 16)] = jnp.maximum(v, 0) + off

    bufs = (buf0, buf1)
    gsems = (gs0, gs1)
    wsems = (ws0, ws1)
    gd = [None, None]
    wd = [None, None]
    gd[0] = pltpu.async_copy(x_hbm.at[idx_v.at[pl.ds(0, CH)]], bufs[0], gsems[0])
    for c in range(NCH):
        cb = c % 2
        nb = (c + 1) % 2
        if c + 1 < NCH:
            if wd[nb] is not None:
                wd[nb].wait()
            gd[nb] = pltpu.async_copy(
                x_hbm.at[idx_v.at[pl.ds((c + 1) * CH, CH)]], bufs[nb], gsems[nb])
        gd[cb].wait()
        wd[cb] = pltpu.async_copy(
            bufs[cb], out_hbm.at[pl.ds(base + c * CH, CH)], wsems[cb])
    wd[0].wait()
    wd[1].wait()


def kernel(x, indices):
    xf = x.reshape(B * S, D)
    idx = indices.reshape(B * L).astype(jnp.int32)
    mesh = plsc.VectorSubcoreMesh(core_axis_name="c", subcore_axis_name="s")
    out = pl.kernel(
        _body,
        mesh=mesh,
        out_type=jax.ShapeDtypeStruct((B * L, D), jnp.float32),
        scratch_types=[
            pltpu.VMEM((RPW,), jnp.int32),
            pltpu.VMEM((CH, D), jnp.float32),
            pltpu.VMEM((CH, D), jnp.float32),
            pltpu.SemaphoreType.DMA,
            pltpu.SemaphoreType.DMA,
            pltpu.SemaphoreType.DMA,
            pltpu.SemaphoreType.DMA,
        ],
    )(xf, idx)
    return out.reshape(B, L, D)


# speedup vs baseline: 1.5655x; 1.0302x over previous
import jax
import jax.numpy as jnp
from jax import lax
from jax.experimental import pallas as pl
from jax.experimental.pallas import tpu as pltpu
from jax.experimental.pallas import tpu_sc as plsc

B, S, D = 4, 4096, 1024
L = 2048
NC, NS = 2, 16
NW = NC * NS
RPW = (B * L) // NW
CH = 32
NCH = RPW // CH


NBUF = 3


def _body(x_hbm, idx_hbm, out_hbm, idx_v, *rest):
    bufs = rest[:NBUF]
    gsems = rest[NBUF:2 * NBUF]
    wsems = rest[2 * NBUF:3 * NBUF]
    wid = lax.axis_index("s") * NC + lax.axis_index("c")
    base = wid * RPW
    off = (base // L) * S

    pltpu.sync_copy(idx_hbm.at[wid], idx_v)
    for j in range(NCH):
        for i in range(CH // 16):
            v = idx_v[j, pl.ds(i * 16, 16)]
            idx_v[j, pl.ds(i * 16, 16)] = jnp.maximum(v, 0) + off

    gd = [None] * NBUF
    wd = [None] * NBUF
    for c in range(NBUF - 1):
        gd[c] = pltpu.async_copy(x_hbm.at[idx_v.at[c]], bufs[c], gsems[c])
    for c in range(NCH):
        cb = c % NBUF
        nx = c + NBUF - 1
        if nx < NCH:
            nb = nx % NBUF
            if wd[nb] is not None:
                wd[nb].wait()
            gd[nb] = pltpu.async_copy(x_hbm.at[idx_v.at[nx]], bufs[nb], gsems[nb])
        gd[cb].wait()
        wd[cb] = pltpu.async_copy(
            bufs[cb], out_hbm.at[pl.ds(base + c * CH, CH)], wsems[cb])
    for c in range(NCH - NBUF, NCH):
        wd[c % NBUF].wait()


def kernel(x, indices):
    xf = x.reshape(B * S, D)
    idx = indices.reshape(NW, NCH, CH).astype(jnp.int32)
    mesh = plsc.VectorSubcoreMesh(core_axis_name="c", subcore_axis_name="s")
    out = pl.kernel(
        _body,
        mesh=mesh,
        out_type=jax.ShapeDtypeStruct((B * L, D), jnp.float32),
        scratch_types=(
            [pltpu.VMEM((NCH, CH), jnp.int32)]
            + [pltpu.VMEM((CH, D), jnp.float32)] * NBUF
            + [pltpu.SemaphoreType.DMA] * (2 * NBUF)
        ),
    )(xf, idx)
    return out.reshape(B, L, D)
